# baseline (speedup 1.0000x reference)
import functools

import jax
import jax.numpy as jnp
from jax import lax
from jax.experimental import pallas as pl
from jax.experimental.pallas import tpu as pltpu
from jax.experimental.pallas import tpu_sc as plsc

D = 2048
KSEL = 256
B = 8
NPIX = B * 32 * 32
SIGMA = 0.1
A = 128
G = 16
LANES = 16
NW = 32
RPW = NPIX // NW
RB = 16
NBLK = RPW // RB


def _select_body(mu_ref, noise_ref, extra_ref, et_ref, topk_ref, scale_ref):
    z = mu_ref[...] + SIGMA * (noise_ref[...] + 0.25 * extra_ref[...])
    gate = jnp.clip(z + 0.5, 0.0, 1.0)
    bits = lax.bitcast_convert_type(gate, jnp.int32)
    bits = jnp.where(bits < 0, 0, bits)

    def bs_step(i, lo):
        cand = lo | (1 << (30 - i))
        cnt = jnp.sum((bits >= cand).astype(jnp.int32))
        return jnp.where(cnt >= KSEL, cand, lo)

    thresh = lax.fori_loop(0, 31, bs_step, jnp.int32(0))
    maskf = (bits >= thresh).astype(jnp.float32)

    ia = lax.broadcasted_iota(jnp.int32, (A, A), 0)
    ja = lax.broadcasted_iota(jnp.int32, (A, A), 1)
    lower = (ja <= ia).astype(jnp.float32)
    colcs = jnp.dot(lower, maskf, preferred_element_type=jnp.float32)
    coltot = colcs[A - 1:A, :]
    ig = lax.broadcasted_iota(jnp.int32, (G, G), 0)
    jg = lax.broadcasted_iota(jnp.int32, (G, G), 1)
    strict = (ig < jg).astype(jnp.float32)
    prefix = jnp.dot(coltot, strict, preferred_element_type=jnp.float32)
    ranks = (colcs + prefix) * maskf

    jlane = lax.broadcasted_iota(jnp.int32, (A, KSEL), 1).astype(jnp.float32)
    arow = lax.broadcasted_iota(jnp.int32, (A, KSEL), 0).astype(jnp.float32)
    topk_acc = jnp.zeros((1, KSEL), jnp.float32)
    scale_acc = jnp.zeros((1, KSEL), jnp.float32)
    for g in range(G):
        rank_col = jnp.broadcast_to(ranks[:, g:g + 1], (A, KSEL))
        gate_col = jnp.broadcast_to(gate[:, g:g + 1], (A, KSEL))
        hit = rank_col == jlane + 1.0
        et_ref[pl.ds(g * A, A), :] = jnp.where(
            hit, gate_col, 0.0).astype(jnp.bfloat16)
        topk_acc += jnp.sum(
            jnp.where(hit, arow + float(g * A), 0.0), axis=0, keepdims=True)
        scale_acc += jnp.sum(
            jnp.where(hit, gate_col, 0.0), axis=0, keepdims=True)
    topk_ref[...] = topk_acc.astype(jnp.int32)
    scale_ref[...] = scale_acc


def _select(mu, noise, extra):
    grid = lambda a: a.reshape(G, A).T
    return pl.pallas_call(
        _select_body,
        out_shape=(
            jax.ShapeDtypeStruct((D, KSEL), jnp.bfloat16),
            jax.ShapeDtypeStruct((1, KSEL), jnp.int32),
            jax.ShapeDtypeStruct((1, KSEL), jnp.float32),
        ),
    )(grid(mu), grid(noise), grid(extra))


def _sc_lane_gather(x2, topk1d, scale1d):
    mesh = plsc.VectorSubcoreMesh(core_axis_name="c", subcore_axis_name="s")

    @functools.partial(
        pl.kernel,
        out_type=jax.ShapeDtypeStruct((NPIX, KSEL), jnp.float32),
        mesh=mesh,
        compiler_params=pltpu.CompilerParams(needs_layout_passes=False),
        scratch_types=[
            pltpu.VMEM((KSEL,), jnp.int32),
            pltpu.VMEM((KSEL,), jnp.float32),
            pltpu.VMEM((2, RB, D), jnp.float32),
            pltpu.VMEM((2, RB, KSEL), jnp.float32),
            pltpu.SemaphoreType.DMA((2,)),
            pltpu.SemaphoreType.DMA((2,)),
        ],
    )
    def k(x_hbm, topk_hbm, scale_hbm, out_hbm, topk_v, scale_v, inb, outb,
          isem, osem):
        wid = lax.axis_index("s") * 2 + lax.axis_index("c")
        base = wid * RPW
        pltpu.sync_copy(topk_hbm, topk_v)
        pltpu.sync_copy(scale_hbm, scale_v)

        def in_copy(g, par):
            return pltpu.make_async_copy(
                x_hbm.at[pl.ds(base + g * RB, RB)], inb.at[par], isem.at[par])

        def out_copy(g, par):
            return pltpu.make_async_copy(
                outb.at[par], out_hbm.at[pl.ds(base + g * RB, RB)],
                osem.at[par])

        in_copy(0, 0).start()

        def block(g, _):
            par = lax.rem(g, 2)
            nxt = lax.rem(g + 1, 2)

            @pl.when(g + 1 < NBLK)
            def _():
                in_copy(g + 1, nxt).start()

            in_copy(g, par).wait()

            @pl.when(g >= 2)
            def _():
                out_copy(g - 2, par).wait()

            def rloop(r, _):
                for c in range(KSEL // LANES):
                    idx16 = topk_v[pl.ds(c * LANES, LANES)]
                    s16 = scale_v[pl.ds(c * LANES, LANES)]
                    v = plsc.load_gather(
                        inb,
                        [jnp.full((LANES,), par, jnp.int32),
                         jnp.full((LANES,), r, jnp.int32),
                         idx16])
                    outb[par, r, pl.ds(c * LANES, LANES)] = v * s16
                return 0

            lax.fori_loop(0, RB, rloop, 0)
            out_copy(g, par).start()
            return 0

        lax.fori_loop(0, NBLK, block, 0)
        out_copy(NBLK - 2, 0).wait()
        out_copy(NBLK - 1, 1).wait()

    return k(x2, topk1d, scale1d)


def kernel(x, mu, noise, extra_noise):
    x2 = x.reshape(B, D, 32, 32).transpose(0, 2, 3, 1).reshape(NPIX, D)
    et, topk, scale = _select(mu, noise, extra_noise)
    out2 = _sc_lane_gather(x2, topk.reshape(KSEL), scale.reshape(KSEL))
    return out2.reshape(B, 32, 32, KSEL).transpose(0, 3, 1, 2)[:, None]


# speedup vs baseline: 1.3905x; 1.3905x over previous
import functools

import jax
import jax.numpy as jnp
from jax import lax
from jax.experimental import pallas as pl
from jax.experimental.pallas import tpu as pltpu
from jax.experimental.pallas import tpu_sc as plsc

D = 2048
KSEL = 256
B = 8
NPIX = B * 32 * 32
SIGMA = 0.1
A = 128
G = 16
LANES = 16
NW = 32
RPW = NPIX // NW
RB = 16
NBLK = RPW // RB


def _select_body(mu_ref, noise_ref, extra_ref, et_ref, topk_ref, scale_ref):
    z = mu_ref[...] + SIGMA * (noise_ref[...] + 0.25 * extra_ref[...])
    gate = jnp.clip(z + 0.5, 0.0, 1.0)
    bits = lax.bitcast_convert_type(gate, jnp.int32)
    bits = jnp.where(bits < 0, 0, bits)

    def bs_step(i, lo):
        cand = lo | (1 << (30 - i))
        cnt = jnp.sum((bits >= cand).astype(jnp.int32))
        return jnp.where(cnt >= KSEL, cand, lo)

    thresh = lax.fori_loop(0, 31, bs_step, jnp.int32(0))
    maskf = (bits >= thresh).astype(jnp.float32)

    ia = lax.broadcasted_iota(jnp.int32, (A, A), 0)
    ja = lax.broadcasted_iota(jnp.int32, (A, A), 1)
    lower = (ja <= ia).astype(jnp.float32)
    colcs = jnp.dot(lower, maskf, preferred_element_type=jnp.float32)
    coltot = colcs[A - 1:A, :]
    ig = lax.broadcasted_iota(jnp.int32, (G, G), 0)
    jg = lax.broadcasted_iota(jnp.int32, (G, G), 1)
    strict = (ig < jg).astype(jnp.float32)
    prefix = jnp.dot(coltot, strict, preferred_element_type=jnp.float32)
    ranks = (colcs + prefix) * maskf

    jlane = lax.broadcasted_iota(jnp.int32, (A, KSEL), 1).astype(jnp.float32)
    arow = lax.broadcasted_iota(jnp.int32, (A, KSEL), 0).astype(jnp.float32)
    topk_acc = jnp.zeros((1, KSEL), jnp.float32)
    scale_acc = jnp.zeros((1, KSEL), jnp.float32)
    for g in range(G):
        rank_col = jnp.broadcast_to(ranks[:, g:g + 1], (A, KSEL))
        gate_col = jnp.broadcast_to(gate[:, g:g + 1], (A, KSEL))
        hit = rank_col == jlane + 1.0
        et_ref[pl.ds(g * A, A), :] = jnp.where(
            hit, gate_col, 0.0).astype(jnp.bfloat16)
        topk_acc += jnp.sum(
            jnp.where(hit, arow + float(g * A), 0.0), axis=0, keepdims=True)
        scale_acc += jnp.sum(
            jnp.where(hit, gate_col, 0.0), axis=0, keepdims=True)
    topk_ref[...] = topk_acc.astype(jnp.int32)
    scale_ref[...] = scale_acc


def _select(mu, noise, extra):
    grid = lambda a: a.reshape(G, A).T
    return pl.pallas_call(
        _select_body,
        out_shape=(
            jax.ShapeDtypeStruct((D, KSEL), jnp.bfloat16),
            jax.ShapeDtypeStruct((1, KSEL), jnp.int32),
            jax.ShapeDtypeStruct((1, KSEL), jnp.float32),
        ),
    )(grid(mu), grid(noise), grid(extra))


def _sc_lane_gather(x2, topk1d, scale1d):
    mesh = plsc.VectorSubcoreMesh(core_axis_name="c", subcore_axis_name="s")

    @functools.partial(
        pl.kernel,
        out_type=jax.ShapeDtypeStruct((NPIX, KSEL), jnp.float32),
        mesh=mesh,
        compiler_params=pltpu.CompilerParams(needs_layout_passes=False),
        scratch_types=[
            pltpu.VMEM((KSEL,), jnp.int32),
            pltpu.VMEM((KSEL,), jnp.float32),
            pltpu.VMEM((2, RB, D), jnp.float32),
            pltpu.VMEM((2, RB, KSEL), jnp.float32),
            pltpu.SemaphoreType.DMA((2,)),
            pltpu.SemaphoreType.DMA((2,)),
        ],
    )
    def k(x_hbm, topk_hbm, scale_hbm, out_hbm, topk_v, scale_v, inb, outb,
          isem, osem):
        wid = lax.axis_index("s") * 2 + lax.axis_index("c")
        base = wid * RPW
        pltpu.sync_copy(topk_hbm, topk_v)
        pltpu.sync_copy(scale_hbm, scale_v)
        idx_regs = [topk_v[pl.ds(c * LANES, LANES)]
                    for c in range(KSEL // LANES)]
        s_regs = [scale_v[pl.ds(c * LANES, LANES)]
                  for c in range(KSEL // LANES)]

        def in_copy(g, par):
            return pltpu.make_async_copy(
                x_hbm.at[pl.ds(base + g * RB, RB)], inb.at[par], isem.at[par])

        def out_copy(g, par):
            return pltpu.make_async_copy(
                outb.at[par], out_hbm.at[pl.ds(base + g * RB, RB)],
                osem.at[par])

        in_copy(0, 0).start()

        def block(g, _):
            par = lax.rem(g, 2)
            nxt = lax.rem(g + 1, 2)

            @pl.when(g + 1 < NBLK)
            def _():
                in_copy(g + 1, nxt).start()

            in_copy(g, par).wait()

            @pl.when(g >= 2)
            def _():
                out_copy(g - 2, par).wait()

            parv = jnp.full((LANES,), par, jnp.int32)
            for r in range(RB):
                rv = jnp.full((LANES,), r, jnp.int32)
                for c in range(KSEL // LANES):
                    v = plsc.load_gather(inb, [parv, rv, idx_regs[c]])
                    outb[par, r, pl.ds(c * LANES, LANES)] = v * s_regs[c]
            out_copy(g, par).start()
            return 0

        lax.fori_loop(0, NBLK, block, 0)
        out_copy(NBLK - 2, 0).wait()
        out_copy(NBLK - 1, 1).wait()

    return k(x2, topk1d, scale1d)


def kernel(x, mu, noise, extra_noise):
    x2 = x.reshape(B, D, 32, 32).transpose(0, 2, 3, 1).reshape(NPIX, D)
    et, topk, scale = _select(mu, noise, extra_noise)
    out2 = _sc_lane_gather(x2, topk.reshape(KSEL), scale.reshape(KSEL))
    return out2.reshape(B, 32, 32, KSEL).transpose(0, 3, 1, 2)[:, None]


# speedup vs baseline: 2.5610x; 1.8418x over previous
import jax
import jax.numpy as jnp
from jax import lax
from jax.experimental import pallas as pl
from jax.experimental.pallas import tpu as pltpu

D = 2048
KSEL = 256
B = 8
NPIX = B * 32 * 32
SIGMA = 0.1

A = 128
G = 16

BLK = 1024


def _build_et(mu, noise, extra, et_ref):
    z = mu + SIGMA * (noise + 0.25 * extra)
    gate = jnp.clip(z + 0.5, 0.0, 1.0)

    bits = lax.bitcast_convert_type(gate, jnp.int32)
    bits = jnp.where(bits < 0, 0, bits)

    def bs_step(i, lo):
        cand = lo | (1 << (30 - i))
        cnt = jnp.sum((bits >= cand).astype(jnp.int32))
        return jnp.where(cnt >= KSEL, cand, lo)

    thresh = lax.fori_loop(0, 31, bs_step, jnp.int32(0))
    maskf = (bits >= thresh).astype(jnp.float32)

    ia = lax.broadcasted_iota(jnp.int32, (A, A), 0)
    ja = lax.broadcasted_iota(jnp.int32, (A, A), 1)
    lower = (ja <= ia).astype(jnp.float32)
    colcs = jnp.dot(lower, maskf, preferred_element_type=jnp.float32)
    coltot = colcs[A - 1:A, :]
    ig = lax.broadcasted_iota(jnp.int32, (G, G), 0)
    jg = lax.broadcasted_iota(jnp.int32, (G, G), 1)
    strict = (ig < jg).astype(jnp.float32)
    prefix = jnp.dot(coltot, strict, preferred_element_type=jnp.float32)
    ranks = (colcs + prefix) * maskf

    jlane = lax.broadcasted_iota(jnp.int32, (A, KSEL), 1).astype(jnp.float32)
    for g in range(G):
        rank_col = jnp.broadcast_to(ranks[:, g:g + 1], (A, KSEL))
        gate_col = jnp.broadcast_to(gate[:, g:g + 1], (A, KSEL))
        hit = rank_col == jlane + 1.0
        et_ref[pl.ds(g * A, A), :] = jnp.where(
            hit, gate_col, 0.0).astype(jnp.bfloat16)


def _fused_body(mu_ref, noise_ref, extra_ref, x_ref, out_ref, et_ref):
    @pl.when(pl.program_id(0) == 0)
    def _():
        _build_et(mu_ref[...], noise_ref[...], extra_ref[...], et_ref)

    out_ref[...] = lax.dot_general(
        x_ref[...].astype(jnp.bfloat16), et_ref[...],
        (((1,), (0,)), ((), ())), preferred_element_type=jnp.float32)


def _fused(mu2, noise2, extra2, x2):
    return pl.pallas_call(
        _fused_body,
        grid=(NPIX // BLK,),
        in_specs=[
            pl.BlockSpec((A, G), lambda i: (0, 0)),
            pl.BlockSpec((A, G), lambda i: (0, 0)),
            pl.BlockSpec((A, G), lambda i: (0, 0)),
            pl.BlockSpec((BLK, D), lambda i: (i, 0)),
        ],
        out_specs=pl.BlockSpec((BLK, KSEL), lambda i: (i, 0)),
        out_shape=jax.ShapeDtypeStruct((NPIX, KSEL), jnp.float32),
        scratch_shapes=[pltpu.VMEM((D, KSEL), jnp.bfloat16)],
    )(mu2, noise2, extra2, x2)


def kernel(x, mu, noise, extra_noise):
    x2 = x.reshape(B, D, 32, 32).transpose(0, 2, 3, 1).reshape(NPIX, D)
    grid = lambda a: a.reshape(G, A).T
    out2 = _fused(grid(mu), grid(noise), grid(extra_noise), x2)
    return out2.reshape(B, 32, 32, KSEL).transpose(0, 3, 1, 2)[:, None]


# speedup vs baseline: 2.5658x; 1.0018x over previous
import jax
import jax.numpy as jnp
from jax import lax
from jax.experimental import pallas as pl
from jax.experimental.pallas import tpu as pltpu

D = 2048
KSEL = 256
B = 8
NPIX = B * 32 * 32
SIGMA = 0.1

A = 128
G = 16

BLK = 2048


def _build_et(mu, noise, extra, et_ref):
    z = mu + SIGMA * (noise + 0.25 * extra)
    gate = jnp.clip(z + 0.5, 0.0, 1.0)

    bits = lax.bitcast_convert_type(gate, jnp.int32)
    bits = jnp.where(bits < 0, 0, bits)

    def bs_step(i, lo):
        cand = lo | (1 << (30 - i))
        cnt = jnp.sum((bits >= cand).astype(jnp.int32))
        return jnp.where(cnt >= KSEL, cand, lo)

    thresh = lax.fori_loop(0, 31, bs_step, jnp.int32(0))
    maskf = (bits >= thresh).astype(jnp.float32)

    ia = lax.broadcasted_iota(jnp.int32, (A, A), 0)
    ja = lax.broadcasted_iota(jnp.int32, (A, A), 1)
    lower = (ja <= ia).astype(jnp.float32)
    colcs = jnp.dot(lower, maskf, preferred_element_type=jnp.float32)
    coltot = colcs[A - 1:A, :]
    ig = lax.broadcasted_iota(jnp.int32, (G, G), 0)
    jg = lax.broadcasted_iota(jnp.int32, (G, G), 1)
    strict = (ig < jg).astype(jnp.float32)
    prefix = jnp.dot(coltot, strict, preferred_element_type=jnp.float32)
    ranks = (colcs + prefix) * maskf

    jlane = lax.broadcasted_iota(jnp.int32, (A, KSEL), 1).astype(jnp.float32)
    for g in range(G):
        rank_col = jnp.broadcast_to(ranks[:, g:g + 1], (A, KSEL))
        gate_col = jnp.broadcast_to(gate[:, g:g + 1], (A, KSEL))
        hit = rank_col == jlane + 1.0
        et_ref[pl.ds(g * A, A), :] = jnp.where(
            hit, gate_col, 0.0).astype(jnp.bfloat16)


def _fused_body(mu_ref, noise_ref, extra_ref, x_ref, out_ref, et_ref):
    @pl.when(pl.program_id(0) == 0)
    def _():
        _build_et(mu_ref[...], noise_ref[...], extra_ref[...], et_ref)

    out_ref[...] = lax.dot_general(
        x_ref[...].astype(jnp.bfloat16), et_ref[...],
        (((1,), (0,)), ((), ())), preferred_element_type=jnp.float32)


def _fused(mu2, noise2, extra2, x2):
    return pl.pallas_call(
        _fused_body,
        grid=(NPIX // BLK,),
        in_specs=[
            pl.BlockSpec((A, G), lambda i: (0, 0)),
            pl.BlockSpec((A, G), lambda i: (0, 0)),
            pl.BlockSpec((A, G), lambda i: (0, 0)),
            pl.BlockSpec((BLK, D), lambda i: (i, 0)),
        ],
        out_specs=pl.BlockSpec((BLK, KSEL), lambda i: (i, 0)),
        out_shape=jax.ShapeDtypeStruct((NPIX, KSEL), jnp.float32),
        scratch_shapes=[pltpu.VMEM((D, KSEL), jnp.bfloat16)],
    )(mu2, noise2, extra2, x2)


def kernel(x, mu, noise, extra_noise):
    x2 = x.reshape(B, D, 32, 32).transpose(0, 2, 3, 1).reshape(NPIX, D)
    grid = lambda a: a.reshape(G, A).T
    out2 = _fused(grid(mu), grid(noise), grid(extra_noise), x2)
    return out2.reshape(B, 32, 32, KSEL).transpose(0, 3, 1, 2)[:, None]
